# baseline (speedup 1.0000x reference)
import functools

import jax
import jax.numpy as jnp
from jax import lax
from jax.experimental import pallas as pl
from jax.experimental.pallas import tpu as pltpu
from jax.experimental.pallas import tpu_sc as plsc

_LAYOUT = [(21, 4.0, 4), (21, 8.0, 4), (21, 16.0, 4), (21, 32.0, 4)]
_N = 1048576
_L = 16
_NW = 32
_P = 256
_PW = _N // _NW
_NCHUNK = _PW // _P
_HASH_P2 = 2654435761 - (1 << 32)
_EPS = 1e-5


def _rsqrt(x):
    i = plsc.bitcast(x, jnp.int32)
    y = plsc.bitcast(jnp.int32(0x5F3759DF) - (i >> 1), jnp.float32)
    hx = x * 0.5
    for _ in range(3):
        y = y * (1.5 - hx * y * y)
    return y


def _body(px_hbm, py_hbm, t0, t1, t2, t3, lw_hbm, out_hbm,
          px_v, py_v, fx_v, fy_v, idx_v, e_v, rows_v, out_v, lw_v, sems,
          semo):
    tables = [t0, t1, t2, t3]
    wid = lax.axis_index("s") * 2 + lax.axis_index("c")
    wbase = wid * _PW
    lane = lax.iota(jnp.int32, _L)

    pltpu.sync_copy(lw_hbm, lw_v)

    def hash_level(lvl, b, cm):
        _hs, cs, _dim = _LAYOUT[lvl]
        mask = jnp.int32((1 << 21) - 1)
        inv_cs = jnp.float32(1.0 / cs)

        def hash_body(g, _):
            o = pl.multiple_of(g * _L, _L)
            po = cm * _P + o
            sx = px_v[pl.ds(po, _L)] * inv_cs
            sy = py_v[pl.ds(po, _L)] * inv_cs
            ix = sx.astype(jnp.int32)
            iy = sy.astype(jnp.int32)
            fx_v[pl.ds(b * _P + o, _L)] = sx - ix.astype(jnp.float32)
            fy_v[pl.ds(b * _P + o, _L)] = sy - iy.astype(jnp.float32)
            p2 = jnp.int32(_HASH_P2)
            hy0 = iy * p2
            hy1 = (iy + 1) * p2
            ix1 = ix + 1
            sl2 = (lane + o) * 2
            for yp, (ha, hb) in enumerate((((ix ^ hy0) & mask,
                                            (ix1 ^ hy0) & mask),
                                           ((ix ^ hy1) & mask,
                                            (ix1 ^ hy1) & mask))):
                qa = ((ha >> 7) << 6) + ((ha >> 3) & 15)
                qb = ((hb >> 7) << 6) + ((hb >> 3) & 15)
                e_v[pl.ds((4 * b + 2 * yp) * _P + o, _L)] = ha & 7
                e_v[pl.ds((4 * b + 2 * yp + 1) * _P + o, _L)] = hb & 7
                for d in range(4):
                    so = (16 * b + 8 * yp + 2 * d) * _P
                    plsc.store_scatter(idx_v, [sl2 + so], qa + (d << 4))
                    plsc.store_scatter(idx_v, [sl2 + (so + 1)], qb + (d << 4))
            return ()

        lax.fori_loop(0, _P // _L, hash_body, (), unroll=False)

    def start_gather(lvl, b):
        return pltpu.async_copy(
            tables[lvl].at[idx_v.at[pl.ds(16 * b * _P, 16 * _P)]],
            rows_v.at[pl.ds(16 * b * _P, 16 * _P)], sems[b])

    def blend_level(lvl, b, ob):
        lw = lw_v[pl.ds(lvl * _L, _L)]

        def blend_body(g, _):
            o = pl.multiple_of(g * _L, _L)
            rows = lane + o
            fx = fx_v[pl.ds(b * _P + o, _L)]
            fy = fy_v[pl.ds(b * _P + o, _L)]
            wx0 = 1.0 - fx
            wy0 = 1.0 - fy
            w00 = wx0 * wy0
            w10 = fx * wy0
            w01 = wx0 * fy
            w11 = fx * fy
            ws = (w00, w10, w01, w11)
            es = [e_v[pl.ds((4 * b + ci) * _P + o, _L)] for ci in range(4)]
            rows2 = rows * 2
            acc = []
            for d in range(4):
                t = None
                for ci in range(4):
                    yp, cx = ci >> 1, ci & 1
                    ro = (16 * b + 8 * yp + 2 * d) * _P + cx
                    f = plsc.load_gather(rows_v, [rows2 + ro, es[ci]])
                    t = ws[ci] * f if t is None else t + ws[ci] * f
                acc.append(t)
            mu = (acc[0] + acc[1] + acc[2] + acc[3]) * 0.25
            c0 = acc[0] - mu
            c1 = acc[1] - mu
            c2 = acc[2] - mu
            c3 = acc[3] - mu
            var = (c0 * c0 + c1 * c1 + c2 * c2 + c3 * c3) * 0.25
            scale = _rsqrt(var + _EPS) * lw
            orows = rows + ob
            for d, cd in enumerate((c0, c1, c2, c3)):
                colo = jnp.full((_L,), lvl * 4 + d, jnp.int32)
                plsc.store_scatter(out_v, [orows, colo], cd * scale)
            return ()

        lax.fori_loop(0, _P // _L, blend_body, (), unroll=False)

    def chunk_body(c, _):
        base = wbase + c * _P
        cm = c % 8

        @pl.when(cm == 0)
        def _():
            pltpu.sync_copy(px_hbm.at[pl.ds(base, 8 * _P)], px_v)
            pltpu.sync_copy(py_hbm.at[pl.ds(base, 8 * _P)], py_v)

        ob = (c & 1) * _P

        @pl.when(c >= 2)
        def _():
            pltpu.make_async_copy(out_v.at[pl.ds(ob, _P)],
                                  out_hbm.at[pl.ds(base, _P)], semo).wait()

        hash_level(0, 0, cm)
        dma = start_gather(0, 0)
        for lvl in range(4):
            nb = (lvl + 1) & 1
            nxt_dma = None
            if lvl + 1 < 4:
                hash_level(lvl + 1, nb, cm)
                nxt_dma = start_gather(lvl + 1, nb)
            dma.wait()
            blend_level(lvl, lvl & 1, ob)
            dma = nxt_dma

        pltpu.async_copy(out_v.at[pl.ds(ob, _P)],
                         out_hbm.at[pl.ds(base, _P)], semo)
        return ()

    lax.fori_loop(0, _NCHUNK, chunk_body, (), unroll=False)

    for tail in (_NCHUNK - 2, _NCHUNK - 1):
        pltpu.make_async_copy(
            out_v.at[pl.ds((tail & 1) * _P, _P)],
            out_hbm.at[pl.ds(wbase + tail * _P, _P)], semo).wait()


@jax.jit
def _run(px, py, t0, t1, t2, t3, lw64):
    mesh = plsc.VectorSubcoreMesh(core_axis_name="c", subcore_axis_name="s")
    return pl.kernel(
        _body,
        out_type=jax.ShapeDtypeStruct((_N, 16), jnp.float32),
        mesh=mesh,
        scratch_types=[
            pltpu.VMEM((8 * _P,), jnp.float32),
            pltpu.VMEM((8 * _P,), jnp.float32),
            pltpu.VMEM((2 * _P,), jnp.float32),
            pltpu.VMEM((2 * _P,), jnp.float32),
            pltpu.VMEM((32 * _P,), jnp.int32),
            pltpu.VMEM((8 * _P,), jnp.int32),
            pltpu.VMEM((32 * _P, 8), jnp.float32),
            pltpu.VMEM((2 * _P, 16), jnp.float32),
            pltpu.VMEM((64,), jnp.float32),
            [pltpu.SemaphoreType.DMA, pltpu.SemaphoreType.DMA],
            pltpu.SemaphoreType.DMA,
        ],
        compiler_params=pltpu.CompilerParams(use_tc_tiling_on_sc=False,
                                             needs_layout_passes=False),
    )(px, py, t0, t1, t2, t3, lw64)


def _native_view(t):
    return (t.reshape(16384, 128, 4).transpose(0, 2, 1).reshape(1048576, 8))


def kernel(positions, table0, table1, table2, table3, level_weights):
    px = positions[:, 0]
    py = positions[:, 1]
    lw64 = jnp.repeat(level_weights, _L)
    return _run(px, py, *(_native_view(t) for t in
                          (table0, table1, table2, table3)), lw64)


# speedup vs baseline: 1.0025x; 1.0025x over previous
import functools

import jax
import jax.numpy as jnp
from jax import lax
from jax.experimental import pallas as pl
from jax.experimental.pallas import tpu as pltpu
from jax.experimental.pallas import tpu_sc as plsc

_LAYOUT = [(21, 4.0, 4), (21, 8.0, 4), (21, 16.0, 4), (21, 32.0, 4)]
_N = 1048576
_L = 16
_NW = 32
_P = 256
_PW = _N // _NW
_NCHUNK = _PW // _P
_HASH_P2 = 2654435761 - (1 << 32)
_EPS = 1e-5


def _rsqrt(x):
    i = plsc.bitcast(x, jnp.int32)
    y = plsc.bitcast(jnp.int32(0x5F3759DF) - (i >> 1), jnp.float32)
    hx = x * 0.5
    for _ in range(3):
        y = y * (1.5 - hx * y * y)
    return y


def _body(px_hbm, py_hbm, t0, t1, t2, t3, lw_hbm, out_hbm,
          px_v, py_v, fx_v, fy_v, idx_v, e_v, rows_v, out_v, lw_v, sems,
          semo):
    tables = [t0, t1, t2, t3]
    wid = lax.axis_index("s") * 2 + lax.axis_index("c")
    wbase = wid * _PW
    lane = lax.iota(jnp.int32, _L)

    pltpu.sync_copy(lw_hbm, lw_v)

    def hash_level(lvl, b, cm):
        _hs, cs, _dim = _LAYOUT[lvl]
        mask = jnp.int32((1 << 21) - 1)
        inv_cs = jnp.float32(1.0 / cs)

        def hash_body(g, _):
            o = pl.multiple_of(g * _L, _L)
            po = cm * _P + o
            sx = px_v[pl.ds(po, _L)] * inv_cs
            sy = py_v[pl.ds(po, _L)] * inv_cs
            ix = sx.astype(jnp.int32)
            iy = sy.astype(jnp.int32)
            fx_v[pl.ds(b * _P + o, _L)] = sx - ix.astype(jnp.float32)
            fy_v[pl.ds(b * _P + o, _L)] = sy - iy.astype(jnp.float32)
            p2 = jnp.int32(_HASH_P2)
            hy0 = iy * p2
            hy1 = (iy + 1) * p2
            ix1 = ix + 1
            for ci, h in enumerate(((ix ^ hy0) & mask,
                                    (ix1 ^ hy0) & mask,
                                    (ix ^ hy1) & mask,
                                    (ix1 ^ hy1) & mask)):
                q0 = ((h >> 7) << 6) + ((h >> 3) & 15)
                e_v[pl.ds((4 * b + ci) * _P + o, _L)] = h & 7
                for d in range(4):
                    idx_v[pl.ds((16 * b + 4 * ci + d) * _P + o, _L)] = (
                        q0 + (d << 4))
            return ()

        lax.fori_loop(0, _P // _L, hash_body, (), unroll=False)

    def start_gather(lvl, b):
        return pltpu.async_copy(
            tables[lvl].at[idx_v.at[pl.ds(16 * b * _P, 16 * _P)]],
            rows_v.at[pl.ds(16 * b * _P, 16 * _P)], sems[b])

    def blend_level(lvl, b, ob):
        lw = lw_v[pl.ds(lvl * _L, _L)]

        def blend_body(g, _):
            o = pl.multiple_of(g * _L, _L)
            rows = lane + o
            fx = fx_v[pl.ds(b * _P + o, _L)]
            fy = fy_v[pl.ds(b * _P + o, _L)]
            wx0 = 1.0 - fx
            wy0 = 1.0 - fy
            w00 = wx0 * wy0
            w10 = fx * wy0
            w01 = wx0 * fy
            w11 = fx * fy
            ws = (w00, w10, w01, w11)
            es = [e_v[pl.ds((4 * b + ci) * _P + o, _L)] for ci in range(4)]
            acc = []
            for d in range(4):
                t = None
                for ci in range(4):
                    f = plsc.load_gather(
                        rows_v, [rows + (16 * b + 4 * ci + d) * _P, es[ci]])
                    t = ws[ci] * f if t is None else t + ws[ci] * f
                acc.append(t)
            mu = (acc[0] + acc[1] + acc[2] + acc[3]) * 0.25
            c0 = acc[0] - mu
            c1 = acc[1] - mu
            c2 = acc[2] - mu
            c3 = acc[3] - mu
            var = (c0 * c0 + c1 * c1 + c2 * c2 + c3 * c3) * 0.25
            scale = _rsqrt(var + _EPS) * lw
            orows = rows + ob
            for d, cd in enumerate((c0, c1, c2, c3)):
                colo = jnp.full((_L,), lvl * 4 + d, jnp.int32)
                plsc.store_scatter(out_v, [orows, colo], cd * scale)
            return ()

        lax.fori_loop(0, _P // _L, blend_body, (), unroll=False)

    def chunk_body(c, _):
        base = wbase + c * _P
        cm = c % 8

        @pl.when(cm == 0)
        def _():
            pltpu.sync_copy(px_hbm.at[pl.ds(base, 8 * _P)], px_v)
            pltpu.sync_copy(py_hbm.at[pl.ds(base, 8 * _P)], py_v)

        ob = (c & 1) * _P

        @pl.when(c >= 2)
        def _():
            pltpu.make_async_copy(out_v.at[pl.ds(ob, _P)],
                                  out_hbm.at[pl.ds(base, _P)], semo).wait()

        hash_level(0, 0, cm)
        dma = start_gather(0, 0)
        for lvl in range(4):
            nb = (lvl + 1) & 1
            nxt_dma = None
            if lvl + 1 < 4:
                hash_level(lvl + 1, nb, cm)
                nxt_dma = start_gather(lvl + 1, nb)
            dma.wait()
            blend_level(lvl, lvl & 1, ob)
            dma = nxt_dma

        pltpu.async_copy(out_v.at[pl.ds(ob, _P)],
                         out_hbm.at[pl.ds(base, _P)], semo)
        return ()

    lax.fori_loop(0, _NCHUNK, chunk_body, (), unroll=False)

    for tail in (_NCHUNK - 2, _NCHUNK - 1):
        pltpu.make_async_copy(
            out_v.at[pl.ds((tail & 1) * _P, _P)],
            out_hbm.at[pl.ds(wbase + tail * _P, _P)], semo).wait()


@jax.jit
def _run(px, py, t0, t1, t2, t3, lw64):
    mesh = plsc.VectorSubcoreMesh(core_axis_name="c", subcore_axis_name="s")
    return pl.kernel(
        _body,
        out_type=jax.ShapeDtypeStruct((_N, 16), jnp.float32),
        mesh=mesh,
        scratch_types=[
            pltpu.VMEM((8 * _P,), jnp.float32),
            pltpu.VMEM((8 * _P,), jnp.float32),
            pltpu.VMEM((2 * _P,), jnp.float32),
            pltpu.VMEM((2 * _P,), jnp.float32),
            pltpu.VMEM((32 * _P,), jnp.int32),
            pltpu.VMEM((8 * _P,), jnp.int32),
            pltpu.VMEM((32 * _P, 8), jnp.float32),
            pltpu.VMEM((2 * _P, 16), jnp.float32),
            pltpu.VMEM((64,), jnp.float32),
            [pltpu.SemaphoreType.DMA, pltpu.SemaphoreType.DMA],
            pltpu.SemaphoreType.DMA,
        ],
        compiler_params=pltpu.CompilerParams(use_tc_tiling_on_sc=False,
                                             needs_layout_passes=False),
    )(px, py, t0, t1, t2, t3, lw64)


def _native_view(t):
    return (t.reshape(16384, 128, 4).transpose(0, 2, 1).reshape(1048576, 8))


def kernel(positions, table0, table1, table2, table3, level_weights):
    px = positions[:, 0]
    py = positions[:, 1]
    lw64 = jnp.repeat(level_weights, _L)
    return _run(px, py, *(_native_view(t) for t in
                          (table0, table1, table2, table3)), lw64)


# speedup vs baseline: 1.0029x; 1.0003x over previous
import functools

import jax
import jax.numpy as jnp
from jax import lax
from jax.experimental import pallas as pl
from jax.experimental.pallas import tpu as pltpu
from jax.experimental.pallas import tpu_sc as plsc

_LAYOUT = [(21, 4.0, 4), (21, 8.0, 4), (21, 16.0, 4), (21, 32.0, 4)]
_N = 1048576
_L = 16
_NW = 32
_P = 256
_PW = _N // _NW
_NCHUNK = _PW // _P
_HASH_P2 = 2654435761 - (1 << 32)
_EPS = 1e-5


def _rsqrt(x):
    i = plsc.bitcast(x, jnp.int32)
    y = plsc.bitcast(jnp.int32(0x5F3759DF) - (i >> 1), jnp.float32)
    hx = x * 0.5
    for _ in range(3):
        y = y * (1.5 - hx * y * y)
    return y


def _body(px_hbm, py_hbm, t0, t1, t2, t3, lw_hbm, out_hbm,
          px_v, py_v, fx_v, fy_v, idx_v, e_v, rows_v, out_v, lw_v, sems,
          semo):
    tables = [t0, t1, t2, t3]
    wid = lax.axis_index("s") * 2 + lax.axis_index("c")
    wbase = wid * _PW
    lane = lax.iota(jnp.int32, _L)

    pltpu.sync_copy(lw_hbm, lw_v)

    def hash_level(lvl, b, cm):
        _hs, cs, _dim = _LAYOUT[lvl]
        mask = jnp.int32((1 << 21) - 1)
        inv_cs = jnp.float32(1.0 / cs)

        def hash_body(g, _):
            o = pl.multiple_of(g * _L, _L)
            po = cm * _P + o
            sx = px_v[pl.ds(po, _L)] * inv_cs
            sy = py_v[pl.ds(po, _L)] * inv_cs
            ix = sx.astype(jnp.int32)
            iy = sy.astype(jnp.int32)
            fx_v[pl.ds(b * _P + o, _L)] = sx - ix.astype(jnp.float32)
            fy_v[pl.ds(b * _P + o, _L)] = sy - iy.astype(jnp.float32)
            p2 = jnp.int32(_HASH_P2)
            hy0 = iy * p2
            hy1 = (iy + 1) * p2
            ix1 = ix + 1
            for ci, h in enumerate(((ix ^ hy0) & mask,
                                    (ix1 ^ hy0) & mask,
                                    (ix ^ hy1) & mask,
                                    (ix1 ^ hy1) & mask)):
                q0 = ((h >> 7) << 6) + ((h >> 3) & 15)
                e_v[pl.ds((4 * b + ci) * _P + o, _L)] = h & 7
                for d in range(4):
                    idx_v[pl.ds((16 * b + 4 * ci + d) * _P + o, _L)] = (
                        q0 + (d << 4))
            return ()

        lax.fori_loop(0, _P // _L, hash_body, (), unroll=False)

    def start_gather(lvl, b):
        h1 = pltpu.async_copy(
            tables[lvl].at[idx_v.at[pl.ds(16 * b * _P, 8 * _P)]],
            rows_v.at[pl.ds(16 * b * _P, 8 * _P)], sems[b])
        h2 = pltpu.async_copy(
            tables[lvl].at[idx_v.at[pl.ds((16 * b + 8) * _P, 8 * _P)]],
            rows_v.at[pl.ds((16 * b + 8) * _P, 8 * _P)], sems[b])
        return (h1, h2)

    def blend_level(lvl, b, ob):
        lw = lw_v[pl.ds(lvl * _L, _L)]

        def blend_body(g, _):
            o = pl.multiple_of(g * _L, _L)
            rows = lane + o
            fx = fx_v[pl.ds(b * _P + o, _L)]
            fy = fy_v[pl.ds(b * _P + o, _L)]
            wx0 = 1.0 - fx
            wy0 = 1.0 - fy
            w00 = wx0 * wy0
            w10 = fx * wy0
            w01 = wx0 * fy
            w11 = fx * fy
            ws = (w00, w10, w01, w11)
            es = [e_v[pl.ds((4 * b + ci) * _P + o, _L)] for ci in range(4)]
            acc = []
            for d in range(4):
                t = None
                for ci in range(4):
                    f = plsc.load_gather(
                        rows_v, [rows + (16 * b + 4 * ci + d) * _P, es[ci]])
                    t = ws[ci] * f if t is None else t + ws[ci] * f
                acc.append(t)
            mu = (acc[0] + acc[1] + acc[2] + acc[3]) * 0.25
            c0 = acc[0] - mu
            c1 = acc[1] - mu
            c2 = acc[2] - mu
            c3 = acc[3] - mu
            var = (c0 * c0 + c1 * c1 + c2 * c2 + c3 * c3) * 0.25
            scale = _rsqrt(var + _EPS) * lw
            orows = rows + ob
            for d, cd in enumerate((c0, c1, c2, c3)):
                colo = jnp.full((_L,), lvl * 4 + d, jnp.int32)
                plsc.store_scatter(out_v, [orows, colo], cd * scale)
            return ()

        lax.fori_loop(0, _P // _L, blend_body, (), unroll=False)

    def chunk_body(c, _):
        base = wbase + c * _P
        cm = c % 8

        @pl.when(cm == 0)
        def _():
            pltpu.sync_copy(px_hbm.at[pl.ds(base, 8 * _P)], px_v)
            pltpu.sync_copy(py_hbm.at[pl.ds(base, 8 * _P)], py_v)

        ob = (c & 1) * _P

        @pl.when(c >= 2)
        def _():
            pltpu.make_async_copy(out_v.at[pl.ds(ob, _P)],
                                  out_hbm.at[pl.ds(base, _P)], semo).wait()

        hash_level(0, 0, cm)
        dma = start_gather(0, 0)
        for lvl in range(4):
            nb = (lvl + 1) & 1
            nxt_dma = None
            if lvl + 1 < 4:
                hash_level(lvl + 1, nb, cm)
                nxt_dma = start_gather(lvl + 1, nb)
            for h in dma:
                h.wait()
            blend_level(lvl, lvl & 1, ob)
            dma = nxt_dma

        pltpu.async_copy(out_v.at[pl.ds(ob, _P)],
                         out_hbm.at[pl.ds(base, _P)], semo)
        return ()

    lax.fori_loop(0, _NCHUNK, chunk_body, (), unroll=False)

    for tail in (_NCHUNK - 2, _NCHUNK - 1):
        pltpu.make_async_copy(
            out_v.at[pl.ds((tail & 1) * _P, _P)],
            out_hbm.at[pl.ds(wbase + tail * _P, _P)], semo).wait()


@jax.jit
def _run(px, py, t0, t1, t2, t3, lw64):
    mesh = plsc.VectorSubcoreMesh(core_axis_name="c", subcore_axis_name="s")
    return pl.kernel(
        _body,
        out_type=jax.ShapeDtypeStruct((_N, 16), jnp.float32),
        mesh=mesh,
        scratch_types=[
            pltpu.VMEM((8 * _P,), jnp.float32),
            pltpu.VMEM((8 * _P,), jnp.float32),
            pltpu.VMEM((2 * _P,), jnp.float32),
            pltpu.VMEM((2 * _P,), jnp.float32),
            pltpu.VMEM((32 * _P,), jnp.int32),
            pltpu.VMEM((8 * _P,), jnp.int32),
            pltpu.VMEM((32 * _P, 8), jnp.float32),
            pltpu.VMEM((2 * _P, 16), jnp.float32),
            pltpu.VMEM((64,), jnp.float32),
            [pltpu.SemaphoreType.DMA, pltpu.SemaphoreType.DMA],
            pltpu.SemaphoreType.DMA,
        ],
        compiler_params=pltpu.CompilerParams(use_tc_tiling_on_sc=False,
                                             needs_layout_passes=False),
    )(px, py, t0, t1, t2, t3, lw64)


def _native_view(t):
    return (t.reshape(16384, 128, 4).transpose(0, 2, 1).reshape(1048576, 8))


def kernel(positions, table0, table1, table2, table3, level_weights):
    px = positions[:, 0]
    py = positions[:, 1]
    lw64 = jnp.repeat(level_weights, _L)
    return _run(px, py, *(_native_view(t) for t in
                          (table0, table1, table2, table3)), lw64)
